# Initial kernel scaffold; baseline (speedup 1.0000x reference)
import jax
import jax.numpy as jnp
from jax.experimental import pallas as pl


def kernel(x, edge_index, edge_weight):
    raise NotImplementedError("write your pallas kernel here")


# speedup vs baseline: 4.4100x; 4.4100x over previous
import jax
import jax.numpy as jnp
from jax import lax
from jax.experimental import pallas as pl
from jax.experimental.pallas import tpu as pltpu
from jax.experimental.pallas import tpu_sc as plsc

C = 1.0
N_NODES = 10000
D_FEAT = 128
N_EDGES = 320000
MIN_NORM = 1e-15
EPS = 4e-3

NC = 2
NS = 16
NW = NC * NS
B = 25
NJ = 8
SB = NJ * B
NROWS = N_EDGES // B
RPW = NROWS // NW
NSB = RPW // NJ
NSC = 2
ZR = 16


def _sc_spmm_body(table_hbm, src_hbm, dst_hbm, w_hbm, out_hbm,
                  src_v, dst_v, w_v, grows_v, srows_v, zero_v, acc_sh,
                  gsem, ssem):
    cid = lax.axis_index("c")
    sid = lax.axis_index("s")
    wid = cid * NS + sid

    def _zfill(r, _):
        for d in range(D_FEAT // 16):
            zero_v[r, pl.ds(d * 16, 16)] = jnp.zeros((16,), jnp.float32)
        return 0
    lax.fori_loop(0, ZR, _zfill, 0)
    nz = jnp.where(sid == NS - 1, 25, 40)

    def _zcopy(k, _):
        off = pl.multiple_of(sid * 640 + k * ZR, 8)
        pltpu.sync_copy(zero_v, acc_sh.at[pl.ds(off, ZR)])
        return 0
    lax.fori_loop(0, nz, _zcopy, 0)
    plsc.subcore_barrier()

    def _super_block(s, _):
        row0 = pl.multiple_of(wid * RPW + s * NJ, 8)
        pltpu.sync_copy(src_hbm.at[pl.ds(row0, NJ)], src_v)
        pltpu.sync_copy(dst_hbm.at[pl.ds(row0, NJ)], dst_v)
        e16 = pl.multiple_of((wid * RPW + s * NJ) * (B * 16), 8)
        pltpu.sync_copy(w_hbm.at[pl.ds(e16, SB * 16)], w_v)
        descs = []
        for j in range(NJ):
            descs.append(pltpu.async_copy(
                table_hbm.at[src_v.at[j]], grows_v.at[j], gsem))
        sdescs = [None] * NSC
        for j in range(NJ):
            descs[j].wait()
            if sdescs[j % NSC] is not None:
                sdescs[j % NSC].wait()

            def _wmul(e, _):
                woff = pl.multiple_of((j * B + e) * 16, 8)
                wsplat = w_v[pl.ds(woff, 16)]
                for d in range(D_FEAT // 16):
                    ds = pl.ds(d * 16, 16)
                    srows_v[j % NSC, e, ds] = grows_v[j, e, ds] * wsplat
                return 0
            lax.fori_loop(0, B, _wmul, 0)
            sdescs[j % NSC] = pltpu.async_copy(
                srows_v.at[j % NSC], acc_sh.at[dst_v.at[j]], ssem, add=True)
        for d in sdescs:
            d.wait()
        return 0

    lax.fori_loop(0, NSB, _super_block, 0)
    plsc.subcore_barrier()

    @pl.when(sid < NS - 1)
    def _dump_main():
        off = pl.multiple_of(sid * 640, 8)
        pltpu.sync_copy(acc_sh.at[pl.ds(off, 640)],
                        out_hbm.at[cid, pl.ds(off, 640)])

    @pl.when(sid == NS - 1)
    def _dump_tail():
        pltpu.sync_copy(acc_sh.at[pl.ds(9600, 400)],
                        out_hbm.at[cid, pl.ds(9600, 400)])


_sc_spmm = pl.kernel(
    _sc_spmm_body,
    out_type=jax.ShapeDtypeStruct((NC, N_NODES, D_FEAT), jnp.float32),
    mesh=plsc.VectorSubcoreMesh(core_axis_name="c", subcore_axis_name="s"),
    scratch_types=[
        pltpu.VMEM((NJ, B), jnp.int32),
        pltpu.VMEM((NJ, B), jnp.int32),
        pltpu.VMEM((SB * 16,), jnp.float32),
        pltpu.VMEM((NJ, B, D_FEAT), jnp.float32),
        pltpu.VMEM((NSC, B, D_FEAT), jnp.float32),
        pltpu.VMEM((ZR, D_FEAT), jnp.float32),
        pltpu.VMEM_SHARED((N_NODES, D_FEAT), jnp.float32),
        pltpu.SemaphoreType.DMA,
        pltpu.SemaphoreType.DMA,
    ],
)


def _logmap_body(x_ref, o_ref):
    x = x_ref[...]
    norm = jnp.maximum(jnp.sqrt(jnp.sum(x * x, axis=-1, keepdims=True)),
                       MIN_NORM)
    z = jnp.clip(norm, -1 + 1e-7, 1 - 1e-7)
    o_ref[...] = 0.5 * jnp.log((1 + z) / (1 - z)) / norm * x


def _add_body(y_ref, o_ref):
    o_ref[...] = y_ref[0] + y_ref[1]


def _expmap_body(y_ref, o_ref):
    u = y_ref[0] + y_ref[1]
    un = jnp.maximum(jnp.sqrt(jnp.sum(u * u, axis=-1, keepdims=True)),
                     MIN_NORM)
    gamma = jnp.tanh(un) * u / un
    gn = jnp.maximum(jnp.sqrt(jnp.sum(gamma * gamma, axis=-1, keepdims=True)),
                     MIN_NORM)
    maxnorm = 1.0 - EPS
    o_ref[...] = jnp.where(gn > maxnorm, gamma / gn * maxnorm, gamma)


_R = 2000


def _logmap_tc(x):
    return pl.pallas_call(
        _logmap_body,
        grid=(N_NODES // _R,),
        in_specs=[pl.BlockSpec((_R, D_FEAT), lambda i: (i, 0))],
        out_specs=pl.BlockSpec((_R, D_FEAT), lambda i: (i, 0)),
        out_shape=jax.ShapeDtypeStruct((N_NODES, D_FEAT), jnp.float32),
    )(x)


def _add_tc(y):
    return pl.pallas_call(
        _add_body,
        grid=(N_NODES // _R,),
        in_specs=[pl.BlockSpec((2, _R, D_FEAT), lambda i: (0, i, 0))],
        out_specs=pl.BlockSpec((_R, D_FEAT), lambda i: (i, 0)),
        out_shape=jax.ShapeDtypeStruct((N_NODES, D_FEAT), jnp.float32),
    )(y)


def _expmap_tc(y):
    return pl.pallas_call(
        _expmap_body,
        grid=(N_NODES // _R,),
        in_specs=[pl.BlockSpec((2, _R, D_FEAT), lambda i: (0, i, 0))],
        out_specs=pl.BlockSpec((_R, D_FEAT), lambda i: (i, 0)),
        out_shape=jax.ShapeDtypeStruct((N_NODES, D_FEAT), jnp.float32),
    )(y)


@jax.jit
def kernel(x, edge_index, edge_weight):
    src = edge_index[0].astype(jnp.int32)
    dst = edge_index[1].astype(jnp.int32)
    src2d = src.reshape(NROWS, B)
    dst2d = dst.reshape(NROWS, B)
    w16 = jnp.broadcast_to(edge_weight[:, None], (N_EDGES, 16)).reshape(-1)
    xt = _logmap_tc(x)
    p = _sc_spmm(xt, src2d, dst2d, w16)
    y = _add_tc(p)
    p = _sc_spmm(y, src2d, dst2d, w16)
    return _expmap_tc(p)
